# baseline (speedup 1.0000x reference)
import jax
import jax.numpy as jnp
from jax.experimental import pallas as pl
from jax.experimental.pallas import tpu as pltpu

_V = 1000
_C = 64
_K = 20
_A = 3.0
_MIX = 0.05
_RB = 200
_TB = 8
_T = 192


def _adj_body(e1b_ref, e2b_ref, e1_ref, e2_ref, w1_ref, b1_ref, w2_ref,
              b2_ref, an_ref):
    i = pl.program_id(0)
    dn_nt = (((1,), (1,)), ((), ()))
    f32 = jnp.float32

    def nv(e, w, b):
        return jnp.tanh(_A * (jax.lax.dot_general(
            e, w, dn_nt, preferred_element_type=f32) + b))

    nv1 = nv(e1_ref[...], w1_ref[...], b1_ref[...])
    nv2 = nv(e2_ref[...], w2_ref[...], b2_ref[...])
    nv1b = nv(e1b_ref[...], w1_ref[...], b1_ref[...])
    nv2b = nv(e2b_ref[...], w2_ref[...], b2_ref[...])
    a = (jax.lax.dot_general(nv1b, nv2, dn_nt, preferred_element_type=f32)
         - jax.lax.dot_general(nv2b, nv1, dn_nt, preferred_element_type=f32))
    adj = jnp.maximum(jnp.tanh(_A * a), 0.0)
    bits = jax.lax.bitcast_convert_type(adj, jnp.int32)

    def step(t, ans):
        cand = ans | jax.lax.shift_left(jnp.int32(1), 29 - t)
        cnt = jnp.sum((bits >= cand).astype(jnp.int32), axis=1, keepdims=True)
        return jnp.where(cnt >= _K, cand, ans)

    ans = jax.lax.fori_loop(0, 30, step, jnp.zeros((_RB, 1), jnp.int32))
    gt = bits > ans
    tie = bits == ans
    cnt_gt = jnp.sum(gt.astype(jnp.int32), axis=1, keepdims=True)
    rowi = jax.lax.broadcasted_iota(jnp.int32, (_V, _V), 0)
    coli = jax.lax.broadcasted_iota(jnp.int32, (_V, _V), 1)
    lt = (rowi < coli).astype(f32)
    prefix = jax.lax.dot_general(tie.astype(f32), lt, (((1,), (0,)), ((), ())),
                                 preferred_element_type=f32)
    keep = gt | (tie & (prefix < (_K - cnt_gt).astype(f32)))
    madj = jnp.where(keep, adj, 0.0)
    bri = jax.lax.broadcasted_iota(jnp.int32, (_RB, _V), 0) + i * _RB
    bci = jax.lax.broadcasted_iota(jnp.int32, (_RB, _V), 1)
    madj = madj + (bri == bci).astype(f32)
    d = jnp.sum(madj, axis=1, keepdims=True)
    an_ref[...] = madj / d


def _prop_body(an_ref, x_ref, w0_ref, w1_ref, w2_ref, bm_ref, y_ref):
    an = an_ref[...]
    dn_tn = (((0,), (0,)), ((), ()))
    f32 = jnp.float32
    for t in range(_TB):
        xt = x_ref[t]
        u = _MIX * xt + (1.0 - _MIX) * jax.lax.dot_general(
            an, xt, dn_tn, preferred_element_type=f32)
        z = _MIX * xt + (1.0 - _MIX) * jax.lax.dot_general(
            an, u, dn_tn, preferred_element_type=f32)
        y_ref[t] = (jnp.dot(xt, w0_ref[...], preferred_element_type=f32)
                    + jnp.dot(u, w1_ref[...], preferred_element_type=f32)
                    + jnp.dot(z, w2_ref[...], preferred_element_type=f32)
                    + bm_ref[...])


def kernel(x, emb1, emb2, W1, b1, W2, b2, Wmlp, bmlp):
    f32 = jnp.float32
    b1r = b1.reshape(1, _C).astype(f32)
    b2r = b2.reshape(1, _C).astype(f32)
    bmr = bmlp.reshape(1, _C).astype(f32)
    w0t = Wmlp[:, :_C].T
    w1t = Wmlp[:, _C:2 * _C].T
    w2t = Wmlp[:, 2 * _C:].T

    an = pl.pallas_call(
        _adj_body,
        grid=(_V // _RB,),
        in_specs=[
            pl.BlockSpec((_RB, _C), lambda i: (i, 0)),
            pl.BlockSpec((_RB, _C), lambda i: (i, 0)),
            pl.BlockSpec((_V, _C), lambda i: (0, 0)),
            pl.BlockSpec((_V, _C), lambda i: (0, 0)),
            pl.BlockSpec((_C, _C), lambda i: (0, 0)),
            pl.BlockSpec((1, _C), lambda i: (0, 0)),
            pl.BlockSpec((_C, _C), lambda i: (0, 0)),
            pl.BlockSpec((1, _C), lambda i: (0, 0)),
        ],
        out_specs=pl.BlockSpec((_RB, _V), lambda i: (i, 0)),
        out_shape=jax.ShapeDtypeStruct((_V, _V), f32),
        compiler_params=pltpu.CompilerParams(
            dimension_semantics=("arbitrary",)),
    )(emb1, emb2, emb1, emb2, W1, b1r, W2, b2r)

    y = pl.pallas_call(
        _prop_body,
        grid=(_T // _TB,),
        in_specs=[
            pl.BlockSpec((_V, _V), lambda i: (0, 0)),
            pl.BlockSpec((_TB, _V, _C), lambda i: (i, 0, 0)),
            pl.BlockSpec((_C, _C), lambda i: (0, 0)),
            pl.BlockSpec((_C, _C), lambda i: (0, 0)),
            pl.BlockSpec((_C, _C), lambda i: (0, 0)),
            pl.BlockSpec((1, _C), lambda i: (0, 0)),
        ],
        out_specs=pl.BlockSpec((_TB, _V, _C), lambda i: (i, 0, 0)),
        out_shape=jax.ShapeDtypeStruct((_T, _V, _C), f32),
        compiler_params=pltpu.CompilerParams(
            dimension_semantics=("arbitrary",)),
    )(an, x, w0t, w1t, w2t, bmr)
    return y


# speedup vs baseline: 2.2283x; 2.2283x over previous
import jax
import jax.numpy as jnp
from jax.experimental import pallas as pl
from jax.experimental.pallas import tpu as pltpu

_V = 1000
_C = 64
_K = 20
_A = 3.0
_MIX = 0.05
_RB = 200
_TB = 8
_TG = 4
_T = 192
_NG = _TG * _C


def _adj_body(e1_ref, e2_ref, w1_ref, b1_ref, w2_ref, b2_ref, anT_ref):
    dn_nt = (((1,), (1,)), ((), ()))
    f32 = jnp.float32

    def nv(e, w, b):
        return jnp.tanh(_A * (jax.lax.dot_general(
            e, w, dn_nt, preferred_element_type=f32) + b))

    nv1 = nv(e1_ref[...], w1_ref[...], b1_ref[...])
    nv2 = nv(e2_ref[...], w2_ref[...], b2_ref[...])
    g = (jax.lax.dot_general(nv2, nv1, dn_nt, preferred_element_type=f32)
         - jax.lax.dot_general(nv1, nv2, dn_nt, preferred_element_type=f32))
    adjt = jnp.maximum(jnp.tanh(_A * g), 0.0)
    bits = jax.lax.bitcast_convert_type(adjt, jnp.int32)

    def step(t, ans):
        cand = ans | jax.lax.shift_left(jnp.int32(1), 29 - t)
        cnt = jnp.sum((bits >= cand).astype(jnp.int32), axis=0, keepdims=True)
        return jnp.where(cnt >= _K, cand, ans)

    ans = jax.lax.fori_loop(0, 30, step, jnp.zeros((1, _V), jnp.int32))
    gt = bits > ans
    tie = bits == ans
    cnt_gt = jnp.sum(gt.astype(jnp.int32), axis=0, keepdims=True)
    rowi = jax.lax.broadcasted_iota(jnp.int32, (_V, _V), 0)
    coli = jax.lax.broadcasted_iota(jnp.int32, (_V, _V), 1)
    ltm = (coli < rowi).astype(f32)
    prefix = jax.lax.dot_general(ltm, tie.astype(f32), (((1,), (0,)), ((), ())),
                                 preferred_element_type=f32)
    keep = gt | (tie & (prefix < (_K - cnt_gt).astype(f32)))
    madj = jnp.where(keep, adjt, 0.0)
    madj = madj + (rowi == coli).astype(f32)
    d = jnp.sum(madj, axis=0, keepdims=True)
    anT_ref[...] = (madj / d).astype(jnp.bfloat16)


def _prop_body(anT_ref, x_ref, w0_ref, w1_ref, w2_ref, bm_ref, y_ref):
    ant = anT_ref[...]
    dn = (((1,), (0,)), ((), ()))
    f32 = jnp.float32
    for g in range(_TB // _TG):
        xg = jnp.concatenate(
            [x_ref[g * _TG + j] for j in range(_TG)], axis=1)
        p = jax.lax.dot_general(ant, xg.astype(jnp.bfloat16), dn,
                                preferred_element_type=f32)
        u = _MIX * xg + (1.0 - _MIX) * p
        q = jax.lax.dot_general(ant, u.astype(jnp.bfloat16), dn,
                                preferred_element_type=f32)
        z = _MIX * xg + (1.0 - _MIX) * q
        y = (jax.lax.dot_general(xg.astype(jnp.bfloat16), w0_ref[...], dn,
                                 preferred_element_type=f32)
             + jax.lax.dot_general(u.astype(jnp.bfloat16), w1_ref[...], dn,
                                   preferred_element_type=f32)
             + jax.lax.dot_general(z.astype(jnp.bfloat16), w2_ref[...], dn,
                                   preferred_element_type=f32)
             + bm_ref[...])
        for j in range(_TG):
            y_ref[g * _TG + j] = y[:, j * _C:(j + 1) * _C]


def kernel(x, emb1, emb2, W1, b1, W2, b2, Wmlp, bmlp):
    f32 = jnp.float32
    bf16 = jnp.bfloat16
    b1r = b1.reshape(1, _C).astype(f32)
    b2r = b2.reshape(1, _C).astype(f32)
    eye4 = jnp.eye(_TG, dtype=f32)
    w0bd = jnp.kron(eye4, Wmlp[:, :_C].T).astype(bf16)
    w1bd = jnp.kron(eye4, Wmlp[:, _C:2 * _C].T).astype(bf16)
    w2bd = jnp.kron(eye4, Wmlp[:, 2 * _C:].T).astype(bf16)
    bmbd = jnp.tile(bmlp.reshape(1, _C), (1, _TG)).astype(f32)

    ant = pl.pallas_call(
        _adj_body,
        out_shape=jax.ShapeDtypeStruct((_V, _V), bf16),
    )(emb1, emb2, W1, b1r, W2, b2r)

    y = pl.pallas_call(
        _prop_body,
        grid=(_T // _TB,),
        in_specs=[
            pl.BlockSpec((_V, _V), lambda i: (0, 0)),
            pl.BlockSpec((_TB, _V, _C), lambda i: (i, 0, 0)),
            pl.BlockSpec((_NG, _NG), lambda i: (0, 0)),
            pl.BlockSpec((_NG, _NG), lambda i: (0, 0)),
            pl.BlockSpec((_NG, _NG), lambda i: (0, 0)),
            pl.BlockSpec((1, _NG), lambda i: (0, 0)),
        ],
        out_specs=pl.BlockSpec((_TB, _V, _C), lambda i: (i, 0, 0)),
        out_shape=jax.ShapeDtypeStruct((_T, _V, _C), f32),
        compiler_params=pltpu.CompilerParams(
            dimension_semantics=("arbitrary",)),
    )(ant, x, w0bd, w1bd, w2bd, bmbd)
    return y
